# Initial kernel scaffold; baseline (speedup 1.0000x reference)
import functools

import jax
import jax.numpy as jnp
import numpy as np
from jax import lax
from jax.experimental import pallas as pl
from jax.experimental.pallas import tpu as pltpu
from jax.experimental.pallas import tpu_sc as plsc

F = 128
S = 10
NUM_RADIAL = 8
R_MAX = 5.0
INV_SQRT_AVG_NEIGH = float(1.0 / np.sqrt(32.0))

NC = 2
NS = 16
CHUNK = 80

EDGE_BLK = 512


def _edge_body(vt, wr1t0, wr2t0, wangt0, wr1t1, wr2t1, wangt1, o0, o1):
    v = vt[...]
    x = v[0:1, :]
    y = v[1:2, :]
    z = v[2:3, :]
    l2 = x * x + y * y + z * z + 1e-18
    lengths = jnp.sqrt(l2)
    inv_len = 1.0 / lengths
    ux = x * inv_len
    uy = y * inv_len
    uz = z * inv_len
    xs = lengths * (1.0 / R_MAX)
    n = lax.broadcasted_iota(jnp.float32, (NUM_RADIAL, 1), 0) + 1.0
    bessel = (np.sqrt(2.0 / R_MAX) * jnp.sin(n * (np.pi * xs))) / (lengths + 1e-9)
    xs5 = (xs * xs) * (xs * xs) * xs
    env = 1.0 + xs5 * (-21.0 + xs * (35.0 - 15.0 * xs))
    env = jnp.where(xs < 1.0, env, 0.0)
    radT = bessel * env
    angT = jnp.concatenate(
        [jnp.ones_like(x), ux, uy, uz,
         ux * ux, ux * uy, ux * uz, uy * uy, uy * uz, uz * uz,
         ux * ux * ux, ux * ux * uy, ux * ux * uz, ux * uy * uy,
         ux * uy * uz, ux * uz * uz, uy * uy * uy, uy * uy * uz,
         uy * uz * uz, uz * uz * uz],
        axis=0)
    for wr1t, wr2t, wangt, o in ((wr1t0, wr2t0, wangt0, o0),
                                 (wr1t1, wr2t1, wangt1, o1)):
        t = jnp.dot(wr1t[...], radT, preferred_element_type=jnp.float32)
        t = t * jax.nn.sigmoid(t)
        rwT = jnp.dot(wr2t[...], t, preferred_element_type=jnp.float32)
        awT = jnp.dot(wangt[...], angT, preferred_element_type=jnp.float32)
        o[...] = (rwT * awT).T


def _edge_weights(vt, p):
    E = vt.shape[1]
    wspec = lambda shp: pl.BlockSpec(shp, lambda i: (0, 0))
    return pl.pallas_call(
        _edge_body,
        grid=(E // EDGE_BLK,),
        in_specs=[
            pl.BlockSpec((8, EDGE_BLK), lambda i: (0, i)),
            wspec((64, NUM_RADIAL)), wspec((F, 64)), wspec((F, 20)),
            wspec((64, NUM_RADIAL)), wspec((F, 64)), wspec((F, 20)),
        ],
        out_specs=[pl.BlockSpec((EDGE_BLK, F), lambda i: (i, 0))] * 2,
        out_shape=[jax.ShapeDtypeStruct((E, F), jnp.float32)] * 2,
    )(vt,
      p["Wr1_0"].T, p["Wr2_0"].T, p["W_ang0"].T,
      p["Wr1_1"].T, p["Wr2_1"].T, p["W_ang1"].T)


def _onehot(spec_ref):
    return (spec_ref[...] == lax.broadcasted_iota(jnp.int32, (1, S), 1)
            ).astype(jnp.float32)


def _node0_body(spec, emb, wlin, hl_out):
    oh = _onehot(spec)
    h = jnp.dot(oh, emb[...], preferred_element_type=jnp.float32)
    hl_out[...] = jnp.dot(h, wlin[...], preferred_element_type=jnp.float32)


def _node0(spec2d, p):
    N = spec2d.shape[0]
    return pl.pallas_call(
        _node0_body,
        out_shape=jax.ShapeDtypeStruct((N, F), jnp.float32),
    )(spec2d, p["emb"], p["W_lin0"])


def _node1_body(aggp, spec, skip, c2t, c3t, wprod, wro, wlin_next,
                h1_out, hl1_out, out0_out):
    agg = (aggp[0] + aggp[1]) * INV_SQRT_AVG_NEIGH
    oh = _onehot(spec)
    agg = agg * jnp.dot(oh, skip[...], preferred_element_type=jnp.float32)
    c2 = jnp.dot(oh, c2t[...], preferred_element_type=jnp.float32)
    c3 = jnp.dot(oh, c3t[...], preferred_element_type=jnp.float32)
    poly = agg * (1.0 + agg * (c2 + c3 * agg))
    h1 = jnp.dot(poly, wprod[...], preferred_element_type=jnp.float32)
    h1_out[...] = h1
    hl1_out[...] = jnp.dot(h1, wlin_next[...], preferred_element_type=jnp.float32)
    out0_out[...] = jnp.dot(h1, wro[...], preferred_element_type=jnp.float32)


def _node1(aggp, spec2d, p):
    N = spec2d.shape[0]
    return pl.pallas_call(
        _node1_body,
        out_shape=[
            jax.ShapeDtypeStruct((N, F), jnp.float32),
            jax.ShapeDtypeStruct((N, F), jnp.float32),
            jax.ShapeDtypeStruct((N, 1), jnp.float32),
        ],
    )(aggp, spec2d, p["skip0"], p["c2_0"], p["c3_0"],
      p["W_prod0"], p["W_ro0"], p["W_lin1"])


def _node2_body(aggp, h1, spec, skip, c2t, c3t, wprod, wmlp, wro, out1_out):
    agg = (aggp[0] + aggp[1]) * INV_SQRT_AVG_NEIGH
    oh = _onehot(spec)
    sc = h1[...] * jnp.dot(oh, skip[...], preferred_element_type=jnp.float32)
    c2 = jnp.dot(oh, c2t[...], preferred_element_type=jnp.float32)
    c3 = jnp.dot(oh, c3t[...], preferred_element_type=jnp.float32)
    poly = agg * (1.0 + agg * (c2 + c3 * agg))
    h2 = jnp.dot(poly, wprod[...], preferred_element_type=jnp.float32) + sc
    zz = jnp.dot(h2, wmlp[...], preferred_element_type=jnp.float32)
    zz = zz * jax.nn.sigmoid(zz)
    out1_out[...] = jnp.dot(zz, wro[...], preferred_element_type=jnp.float32)


def _node2(aggp, h1, spec2d, p):
    N = spec2d.shape[0]
    return pl.pallas_call(
        _node2_body,
        out_shape=jax.ShapeDtypeStruct((N, 1), jnp.float32),
    )(aggp, h1, spec2d, p["skip1"], p["c2_1"], p["c3_1"],
      p["W_prod1"], p["W_mlp1"], p["W_ro1"])


def _sc_gather_scatter(hl, send2d, recv2d, rwaw):
    N = hl.shape[0]
    E = rwaw.shape[0]
    NW = NC * NS
    per_w = E // NW
    assert per_w * NW == E and per_w % CHUNK == 0
    n_chunks = per_w // CHUNK
    rows_per_tile = N // NS
    assert rows_per_tile * NS == N

    mesh = plsc.VectorSubcoreMesh(core_axis_name="c", subcore_axis_name="s",
                                  num_cores=NC, num_subcores=NS)

    @functools.partial(
        pl.kernel,
        out_type=jax.ShapeDtypeStruct((NC, N, F), jnp.float32),
        mesh=mesh,
        scratch_types=[
            pltpu.VMEM((n_chunks, CHUNK), jnp.int32),
            pltpu.VMEM((n_chunks, CHUNK), jnp.int32),
            pltpu.VMEM((CHUNK, F), jnp.float32),
            pltpu.VMEM((CHUNK, F), jnp.float32),
            pltpu.VMEM_SHARED((N, F), jnp.float32),
            pltpu.SemaphoreType.DMA,
        ],
    )
    def k(hl_hbm, send_hbm, recv_hbm, rwaw_hbm, out_hbm,
          sidx, ridx, rows, wbuf, acc, sem):
        c = lax.axis_index("c")
        s = lax.axis_index("s")
        gid = c * NS + s

        zero = jnp.zeros((16,), jnp.float32)

        def zbody(i, carry):
            for j in range(F // 16):
                rows[i, pl.ds(j * 16, 16)] = zero
            return carry

        lax.fori_loop(0, CHUNK, zbody, 0)
        r0 = s * rows_per_tile
        full, rem = divmod(rows_per_tile, CHUNK)
        for t in range(full):
            pltpu.sync_copy(rows, acc.at[pl.ds(r0 + t * CHUNK, CHUNK)])
        if rem:
            pltpu.sync_copy(rows.at[pl.ds(0, rem)],
                            acc.at[pl.ds(r0 + full * CHUNK, rem)])

        row_base = gid * n_chunks
        pltpu.sync_copy(send_hbm.at[pl.ds(row_base, n_chunks)], sidx)
        pltpu.sync_copy(recv_hbm.at[pl.ds(row_base, n_chunks)], ridx)
        plsc.subcore_barrier()

        edge_base = gid * per_w

        def chunk_body(i, carry):
            base = edge_base + i * CHUNK
            pltpu.async_copy(hl_hbm.at[sidx.at[i]], rows, sem).wait()
            pltpu.sync_copy(rwaw_hbm.at[pl.ds(base, CHUNK)], wbuf)

            def mbody(r, cc):
                for j in range(F // 16):
                    sl = pl.ds(j * 16, 16)
                    rows[r, sl] = rows[r, sl] * wbuf[r, sl]
                return cc

            lax.fori_loop(0, CHUNK, mbody, 0)
            pltpu.sync_copy(rows, acc.at[ridx.at[i]], add=True)
            return carry

        lax.fori_loop(0, n_chunks, chunk_body, 0)
        plsc.subcore_barrier()

        pltpu.sync_copy(acc.at[pl.ds(r0, rows_per_tile)],
                        out_hbm.at[c].at[pl.ds(r0, rows_per_tile)])

    return k(hl, send2d, recv2d, rwaw)


def kernel(vectors, node_specie, senders, receivers, params):
    N = node_specie.shape[0]
    E = senders.shape[0]
    p = params

    vt = jnp.zeros((8, E), jnp.float32).at[0:3, :].set(vectors.T)
    spec2d = node_specie.reshape(N, 1).astype(jnp.int32)
    send2d = senders.astype(jnp.int32).reshape(E // CHUNK, CHUNK)
    recv2d = receivers.astype(jnp.int32).reshape(E // CHUNK, CHUNK)

    rwaw0, rwaw1 = _edge_weights(vt, p)
    hl0 = _node0(spec2d, p)
    aggp0 = _sc_gather_scatter(hl0, send2d, recv2d, rwaw0)
    h1, hl1, out0 = _node1(aggp0, spec2d, p)
    aggp1 = _sc_gather_scatter(hl1, send2d, recv2d, rwaw1)
    out1 = _node2(aggp1, h1, spec2d, p)
    return jnp.stack([out0, out1], axis=1)


# speedup vs baseline: 2.2432x; 2.2432x over previous
import functools

import jax
import jax.numpy as jnp
import numpy as np
from jax import lax
from jax.experimental import pallas as pl
from jax.experimental.pallas import tpu as pltpu
from jax.experimental.pallas import tpu_sc as plsc

F = 128
S = 10
NUM_RADIAL = 8
R_MAX = 5.0
INV_SQRT_AVG_NEIGH = float(1.0 / np.sqrt(32.0))

NC = 2
NS = 16
CHUNK = 80

EDGE_BLK = 512


def _edge_body(vt, wr1t0, wr2t0, wangt0, wr1t1, wr2t1, wangt1, o0, o1):
    v = vt[...]
    x = v[0:1, :]
    y = v[1:2, :]
    z = v[2:3, :]
    l2 = x * x + y * y + z * z + 1e-18
    lengths = jnp.sqrt(l2)
    inv_len = 1.0 / lengths
    ux = x * inv_len
    uy = y * inv_len
    uz = z * inv_len
    xs = lengths * (1.0 / R_MAX)
    n = (lax.broadcasted_iota(jnp.int32, (NUM_RADIAL, 1), 0) + 1
         ).astype(jnp.float32)
    bessel = (np.sqrt(2.0 / R_MAX) * jnp.sin(n * (np.pi * xs))) / (lengths + 1e-9)
    xs5 = (xs * xs) * (xs * xs) * xs
    env = 1.0 + xs5 * (-21.0 + xs * (35.0 - 15.0 * xs))
    env = jnp.where(xs < 1.0, env, 0.0)
    radT = bessel * env
    angT = jnp.concatenate(
        [jnp.ones_like(x), ux, uy, uz,
         ux * ux, ux * uy, ux * uz, uy * uy, uy * uz, uz * uz,
         ux * ux * ux, ux * ux * uy, ux * ux * uz, ux * uy * uy,
         ux * uy * uz, ux * uz * uz, uy * uy * uy, uy * uy * uz,
         uy * uz * uz, uz * uz * uz],
        axis=0)
    for wr1t, wr2t, wangt, o in ((wr1t0, wr2t0, wangt0, o0),
                                 (wr1t1, wr2t1, wangt1, o1)):
        t = jnp.dot(wr1t[...], radT, preferred_element_type=jnp.float32)
        t = t * jax.nn.sigmoid(t)
        rwT = jnp.dot(wr2t[...], t, preferred_element_type=jnp.float32)
        awT = jnp.dot(wangt[...], angT, preferred_element_type=jnp.float32)
        o[...] = (rwT * awT).T


def _edge_weights(vt, p):
    E = vt.shape[1]
    wspec = lambda shp: pl.BlockSpec(shp, lambda i: (0, 0))
    return pl.pallas_call(
        _edge_body,
        grid=(E // EDGE_BLK,),
        in_specs=[
            pl.BlockSpec((8, EDGE_BLK), lambda i: (0, i)),
            wspec((64, NUM_RADIAL)), wspec((F, 64)), wspec((F, 20)),
            wspec((64, NUM_RADIAL)), wspec((F, 64)), wspec((F, 20)),
        ],
        out_specs=[pl.BlockSpec((EDGE_BLK, F), lambda i: (i, 0))] * 2,
        out_shape=[jax.ShapeDtypeStruct((E, F), jnp.float32)] * 2,
    )(vt,
      p["Wr1_0"].T, p["Wr2_0"].T, p["W_ang0"].T,
      p["Wr1_1"].T, p["Wr2_1"].T, p["W_ang1"].T)


def _onehot(spec_ref):
    return (spec_ref[...] == lax.broadcasted_iota(jnp.int32, (1, S), 1)
            ).astype(jnp.float32)


def _node0_body(spec, emb, wlin, hl_out):
    oh = _onehot(spec)
    h = jnp.dot(oh, emb[...], preferred_element_type=jnp.float32)
    hl_out[...] = jnp.dot(h, wlin[...], preferred_element_type=jnp.float32)


def _node0(spec2d, p):
    N = spec2d.shape[0]
    return pl.pallas_call(
        _node0_body,
        out_shape=jax.ShapeDtypeStruct((N, F), jnp.float32),
    )(spec2d, p["emb"], p["W_lin0"])


def _node1_body(aggp, spec, skip, c2t, c3t, wprod, wro, wlin_next,
                h1_out, hl1_out, out0_out):
    agg = (aggp[0] + aggp[1]) * INV_SQRT_AVG_NEIGH
    oh = _onehot(spec)
    agg = agg * jnp.dot(oh, skip[...], preferred_element_type=jnp.float32)
    c2 = jnp.dot(oh, c2t[...], preferred_element_type=jnp.float32)
    c3 = jnp.dot(oh, c3t[...], preferred_element_type=jnp.float32)
    poly = agg * (1.0 + agg * (c2 + c3 * agg))
    h1 = jnp.dot(poly, wprod[...], preferred_element_type=jnp.float32)
    h1_out[...] = h1
    hl1_out[...] = jnp.dot(h1, wlin_next[...], preferred_element_type=jnp.float32)
    out0_out[...] = jnp.dot(h1, wro[...], preferred_element_type=jnp.float32)


def _node1(aggp, spec2d, p):
    N = spec2d.shape[0]
    return pl.pallas_call(
        _node1_body,
        out_shape=[
            jax.ShapeDtypeStruct((N, F), jnp.float32),
            jax.ShapeDtypeStruct((N, F), jnp.float32),
            jax.ShapeDtypeStruct((N, 1), jnp.float32),
        ],
    )(aggp, spec2d, p["skip0"], p["c2_0"], p["c3_0"],
      p["W_prod0"], p["W_ro0"], p["W_lin1"])


def _node2_body(aggp, h1, spec, skip, c2t, c3t, wprod, wmlp, wro, out1_out):
    agg = (aggp[0] + aggp[1]) * INV_SQRT_AVG_NEIGH
    oh = _onehot(spec)
    sc = h1[...] * jnp.dot(oh, skip[...], preferred_element_type=jnp.float32)
    c2 = jnp.dot(oh, c2t[...], preferred_element_type=jnp.float32)
    c3 = jnp.dot(oh, c3t[...], preferred_element_type=jnp.float32)
    poly = agg * (1.0 + agg * (c2 + c3 * agg))
    h2 = jnp.dot(poly, wprod[...], preferred_element_type=jnp.float32) + sc
    zz = jnp.dot(h2, wmlp[...], preferred_element_type=jnp.float32)
    zz = zz * jax.nn.sigmoid(zz)
    out1_out[...] = jnp.dot(zz, wro[...], preferred_element_type=jnp.float32)


def _node2(aggp, h1, spec2d, p):
    N = spec2d.shape[0]
    return pl.pallas_call(
        _node2_body,
        out_shape=jax.ShapeDtypeStruct((N, 1), jnp.float32),
    )(aggp, h1, spec2d, p["skip1"], p["c2_1"], p["c3_1"],
      p["W_prod1"], p["W_mlp1"], p["W_ro1"])


def _sc_gather_scatter(hl, send1d, recv1d, rwaw):
    N = hl.shape[0]
    E = rwaw.shape[0]
    NW = NC * NS
    per_w = E // NW
    assert per_w * NW == E and per_w % CHUNK == 0
    n_chunks = per_w // CHUNK
    ra = (N // NS) & ~7
    tail = N - ra * NS
    assert tail % 8 == 0 and 0 <= tail

    mesh = plsc.VectorSubcoreMesh(core_axis_name="c", subcore_axis_name="s",
                                  num_cores=NC, num_subcores=NS)

    @functools.partial(
        pl.kernel,
        out_type=jax.ShapeDtypeStruct((NC, N, F), jnp.float32),
        mesh=mesh,
        scratch_types=[
            pltpu.VMEM((CHUNK,), jnp.int32),
            pltpu.VMEM((CHUNK,), jnp.int32),
            pltpu.VMEM((CHUNK, F), jnp.float32),
            pltpu.VMEM((CHUNK, F), jnp.float32),
            pltpu.VMEM_SHARED((N, F), jnp.float32),
            pltpu.SemaphoreType.DMA,
        ],
    )
    def k(hl_hbm, send_hbm, recv_hbm, rwaw_hbm, out_hbm,
          sidx, ridx, rows, wbuf, acc, sem):
        c = lax.axis_index("c")
        s = lax.axis_index("s")
        gid = c * NS + s

        zero = jnp.zeros((16,), jnp.float32)

        def zbody(i, carry):
            for j in range(F // 16):
                rows[i, pl.ds(j * 16, 16)] = zero
            return carry

        lax.fori_loop(0, CHUNK, zbody, 0)
        r0 = s * ra
        full, rem = divmod(ra, CHUNK)
        for t in range(full):
            pltpu.sync_copy(rows, acc.at[pl.ds(r0 + t * CHUNK, CHUNK)])
        if rem:
            pltpu.sync_copy(rows.at[pl.ds(0, rem)],
                            acc.at[pl.ds(r0 + full * CHUNK, rem)])
        if tail:
            @pl.when(s == NS - 1)
            def _zero_tail():
                pltpu.sync_copy(rows.at[pl.ds(0, tail)],
                                acc.at[pl.ds(N - tail, tail)])

        plsc.subcore_barrier()

        edge_base = gid * per_w

        def chunk_body(i, carry):
            base = edge_base + i * CHUNK
            pltpu.sync_copy(send_hbm.at[pl.ds(base, CHUNK)], sidx)
            pltpu.sync_copy(recv_hbm.at[pl.ds(base, CHUNK)], ridx)
            pltpu.async_copy(hl_hbm.at[sidx], rows, sem).wait()
            pltpu.sync_copy(rwaw_hbm.at[pl.ds(base, CHUNK)], wbuf)

            def mbody(r, cc):
                for j in range(F // 16):
                    sl = pl.ds(j * 16, 16)
                    rows[r, sl] = rows[r, sl] * wbuf[r, sl]
                return cc

            lax.fori_loop(0, CHUNK, mbody, 0)
            pltpu.sync_copy(rows, acc.at[ridx], add=True)
            return carry

        lax.fori_loop(0, n_chunks, chunk_body, 0)
        plsc.subcore_barrier()

        pltpu.sync_copy(acc.at[pl.ds(r0, ra)],
                        out_hbm.at[c].at[pl.ds(r0, ra)])
        if tail:
            @pl.when(s == NS - 1)
            def _out_tail():
                pltpu.sync_copy(acc.at[pl.ds(N - tail, tail)],
                                out_hbm.at[c].at[pl.ds(N - tail, tail)])

    return k(hl, send1d, recv1d, rwaw)


def kernel(vectors, node_specie, senders, receivers, params):
    N = node_specie.shape[0]
    E = senders.shape[0]
    p = params

    vt = jnp.zeros((8, E), jnp.float32).at[0:3, :].set(vectors.T)
    spec2d = node_specie.reshape(N, 1).astype(jnp.int32)
    send1d = senders.astype(jnp.int32)
    recv1d = receivers.astype(jnp.int32)

    rwaw0, rwaw1 = _edge_weights(vt, p)
    hl0 = _node0(spec2d, p)
    aggp0 = _sc_gather_scatter(hl0, send1d, recv1d, rwaw0)
    h1, hl1, out0 = _node1(aggp0, spec2d, p)
    aggp1 = _sc_gather_scatter(hl1, send1d, recv1d, rwaw1)
    out1 = _node2(aggp1, h1, spec2d, p)
    return jnp.stack([out0, out1], axis=1)


# speedup vs baseline: 3.7869x; 1.6882x over previous
import functools

import jax
import jax.numpy as jnp
import numpy as np
from jax import lax
from jax.experimental import pallas as pl
from jax.experimental.pallas import tpu as pltpu
from jax.experimental.pallas import tpu_sc as plsc

F = 128
S = 10
NUM_RADIAL = 8
R_MAX = 5.0
INV_SQRT_AVG_NEIGH = float(1.0 / np.sqrt(32.0))

NC = 2
NS = 16
CHUNK = 80

EDGE_BLK = 512


def _edge_body(vt, wr1t0, wr2t0, wangt0, wr1t1, wr2t1, wangt1, o0, o1):
    v = vt[...]
    x = v[0:1, :]
    y = v[1:2, :]
    z = v[2:3, :]
    l2 = x * x + y * y + z * z + 1e-18
    lengths = jnp.sqrt(l2)
    inv_len = 1.0 / lengths
    ux = x * inv_len
    uy = y * inv_len
    uz = z * inv_len
    xs = lengths * (1.0 / R_MAX)
    n = (lax.broadcasted_iota(jnp.int32, (NUM_RADIAL, 1), 0) + 1
         ).astype(jnp.float32)
    bessel = (np.sqrt(2.0 / R_MAX) * jnp.sin(n * (np.pi * xs))) / (lengths + 1e-9)
    xs5 = (xs * xs) * (xs * xs) * xs
    env = 1.0 + xs5 * (-21.0 + xs * (35.0 - 15.0 * xs))
    env = jnp.where(xs < 1.0, env, 0.0)
    radT = bessel * env
    angT = jnp.concatenate(
        [jnp.ones_like(x), ux, uy, uz,
         ux * ux, ux * uy, ux * uz, uy * uy, uy * uz, uz * uz,
         ux * ux * ux, ux * ux * uy, ux * ux * uz, ux * uy * uy,
         ux * uy * uz, ux * uz * uz, uy * uy * uy, uy * uy * uz,
         uy * uz * uz, uz * uz * uz],
        axis=0)
    for wr1t, wr2t, wangt, o in ((wr1t0, wr2t0, wangt0, o0),
                                 (wr1t1, wr2t1, wangt1, o1)):
        t = jnp.dot(wr1t[...], radT, preferred_element_type=jnp.float32)
        t = t * jax.nn.sigmoid(t)
        rwT = jnp.dot(wr2t[...], t, preferred_element_type=jnp.float32)
        awT = jnp.dot(wangt[...], angT, preferred_element_type=jnp.float32)
        o[...] = (rwT * awT).T


def _edge_weights(vt, p):
    E = vt.shape[1]
    wspec = lambda shp: pl.BlockSpec(shp, lambda i: (0, 0))
    return pl.pallas_call(
        _edge_body,
        grid=(E // EDGE_BLK,),
        in_specs=[
            pl.BlockSpec((8, EDGE_BLK), lambda i: (0, i)),
            wspec((64, NUM_RADIAL)), wspec((F, 64)), wspec((F, 20)),
            wspec((64, NUM_RADIAL)), wspec((F, 64)), wspec((F, 20)),
        ],
        out_specs=[pl.BlockSpec((EDGE_BLK, F), lambda i: (i, 0))] * 2,
        out_shape=[jax.ShapeDtypeStruct((E, F), jnp.float32)] * 2,
    )(vt,
      p["Wr1_0"].T, p["Wr2_0"].T, p["W_ang0"].T,
      p["Wr1_1"].T, p["Wr2_1"].T, p["W_ang1"].T)


def _onehot(spec_ref):
    return (spec_ref[...] == lax.broadcasted_iota(jnp.int32, (1, S), 1)
            ).astype(jnp.float32)


def _node0_body(spec, emb, wlin, hl_out):
    oh = _onehot(spec)
    h = jnp.dot(oh, emb[...], preferred_element_type=jnp.float32)
    hl_out[...] = jnp.dot(h, wlin[...], preferred_element_type=jnp.float32)


def _node0(spec2d, p):
    N = spec2d.shape[0]
    return pl.pallas_call(
        _node0_body,
        out_shape=jax.ShapeDtypeStruct((N, F), jnp.float32),
    )(spec2d, p["emb"], p["W_lin0"])


def _node1_body(aggp, spec, skip, c2t, c3t, wprod, wro, wlin_next,
                h1_out, hl1_out, out0_out):
    agg = (aggp[0] + aggp[1]) * INV_SQRT_AVG_NEIGH
    oh = _onehot(spec)
    agg = agg * jnp.dot(oh, skip[...], preferred_element_type=jnp.float32)
    c2 = jnp.dot(oh, c2t[...], preferred_element_type=jnp.float32)
    c3 = jnp.dot(oh, c3t[...], preferred_element_type=jnp.float32)
    poly = agg * (1.0 + agg * (c2 + c3 * agg))
    h1 = jnp.dot(poly, wprod[...], preferred_element_type=jnp.float32)
    h1_out[...] = h1
    hl1_out[...] = jnp.dot(h1, wlin_next[...], preferred_element_type=jnp.float32)
    out0_out[...] = jnp.dot(h1, wro[...], preferred_element_type=jnp.float32)


def _node1(aggp, spec2d, p):
    N = spec2d.shape[0]
    return pl.pallas_call(
        _node1_body,
        out_shape=[
            jax.ShapeDtypeStruct((N, F), jnp.float32),
            jax.ShapeDtypeStruct((N, F), jnp.float32),
            jax.ShapeDtypeStruct((N, 1), jnp.float32),
        ],
    )(aggp, spec2d, p["skip0"], p["c2_0"], p["c3_0"],
      p["W_prod0"], p["W_ro0"], p["W_lin1"])


def _node2_body(aggp, h1, spec, skip, c2t, c3t, wprod, wmlp, wro, out1_out):
    agg = (aggp[0] + aggp[1]) * INV_SQRT_AVG_NEIGH
    oh = _onehot(spec)
    sc = h1[...] * jnp.dot(oh, skip[...], preferred_element_type=jnp.float32)
    c2 = jnp.dot(oh, c2t[...], preferred_element_type=jnp.float32)
    c3 = jnp.dot(oh, c3t[...], preferred_element_type=jnp.float32)
    poly = agg * (1.0 + agg * (c2 + c3 * agg))
    h2 = jnp.dot(poly, wprod[...], preferred_element_type=jnp.float32) + sc
    zz = jnp.dot(h2, wmlp[...], preferred_element_type=jnp.float32)
    zz = zz * jax.nn.sigmoid(zz)
    out1_out[...] = jnp.dot(zz, wro[...], preferred_element_type=jnp.float32)


def _node2(aggp, h1, spec2d, p):
    N = spec2d.shape[0]
    return pl.pallas_call(
        _node2_body,
        out_shape=jax.ShapeDtypeStruct((N, 1), jnp.float32),
    )(aggp, h1, spec2d, p["skip1"], p["c2_1"], p["c3_1"],
      p["W_prod1"], p["W_mlp1"], p["W_ro1"])


def _sc_gather_scatter(hl, send1d, recv1d, rwaw):
    N = hl.shape[0]
    E = rwaw.shape[0]
    NW = NC * NS
    per_w = E // NW
    assert per_w * NW == E and per_w % CHUNK == 0
    n_chunks = per_w // CHUNK
    ra = (N // NS) & ~7
    tail = N - ra * NS
    assert tail % 8 == 0 and 0 <= tail

    mesh = plsc.VectorSubcoreMesh(core_axis_name="c", subcore_axis_name="s",
                                  num_cores=NC, num_subcores=NS)

    @functools.partial(
        pl.kernel,
        out_type=jax.ShapeDtypeStruct((NC, N, F), jnp.float32),
        mesh=mesh,
        scratch_types=[
            pltpu.VMEM((CHUNK,), jnp.int32),
            pltpu.VMEM((CHUNK,), jnp.int32),
            pltpu.VMEM((CHUNK,), jnp.int32),
            pltpu.VMEM((CHUNK,), jnp.int32),
            pltpu.VMEM((CHUNK, F), jnp.float32),
            pltpu.VMEM((CHUNK, F), jnp.float32),
            pltpu.VMEM((CHUNK, F), jnp.float32),
            pltpu.VMEM((CHUNK, F), jnp.float32),
            pltpu.VMEM_SHARED((N, F), jnp.float32),
            pltpu.SemaphoreType.DMA,
            pltpu.SemaphoreType.DMA,
            pltpu.SemaphoreType.DMA,
            pltpu.SemaphoreType.DMA,
            pltpu.SemaphoreType.DMA,
            pltpu.SemaphoreType.DMA,
            pltpu.SemaphoreType.DMA,
            pltpu.SemaphoreType.DMA,
            pltpu.SemaphoreType.DMA,
            pltpu.SemaphoreType.DMA,
        ],
    )
    def k(hl_hbm, send_hbm, recv_hbm, rwaw_hbm, out_hbm,
          sidx0, sidx1, ridx0, ridx1, rows0, rows1, wbuf0, wbuf1, acc,
          si0, si1, sr0, sr1, sg0, sg1, sw0, sw1, ss0, ss1):
        c = lax.axis_index("c")
        s = lax.axis_index("s")
        gid = c * NS + s

        sidx = (sidx0, sidx1)
        ridx = (ridx0, ridx1)
        rows = (rows0, rows1)
        wbuf = (wbuf0, wbuf1)
        sem_i = (si0, si1)
        sem_r = (sr0, sr1)
        sem_g = (sg0, sg1)
        sem_w = (sw0, sw1)
        sem_s = (ss0, ss1)

        zero = jnp.zeros((16,), jnp.float32)

        def zbody(i, carry):
            for j in range(F // 16):
                rows0[i, pl.ds(j * 16, 16)] = zero
            return carry

        lax.fori_loop(0, CHUNK, zbody, 0)
        r0 = s * ra
        full, rem = divmod(ra, CHUNK)
        for t in range(full):
            pltpu.sync_copy(rows0, acc.at[pl.ds(r0 + t * CHUNK, CHUNK)])
        if rem:
            pltpu.sync_copy(rows0.at[pl.ds(0, rem)],
                            acc.at[pl.ds(r0 + full * CHUNK, rem)])
        if tail:
            @pl.when(s == NS - 1)
            def _zero_tail():
                pltpu.sync_copy(rows0.at[pl.ds(0, tail)],
                                acc.at[pl.ds(N - tail, tail)])

        plsc.subcore_barrier()

        edge_base = gid * per_w
        n = n_chunks

        def issue_sidx(j, b):
            base = edge_base + j * CHUNK
            pltpu.async_copy(send_hbm.at[pl.ds(base, CHUNK)], sidx[b], sem_i[b])

        def wait_sidx(b):
            pltpu.make_async_copy(send_hbm.at[pl.ds(0, CHUNK)], sidx[b],
                                  sem_i[b]).wait()

        def issue_ridx(j, b):
            base = edge_base + j * CHUNK
            pltpu.async_copy(recv_hbm.at[pl.ds(base, CHUNK)], ridx[b], sem_r[b])

        def wait_ridx(b):
            pltpu.make_async_copy(recv_hbm.at[pl.ds(0, CHUNK)], ridx[b],
                                  sem_r[b]).wait()

        def issue_gw(j, b):
            base = edge_base + j * CHUNK
            pltpu.async_copy(hl_hbm.at[sidx[b]], rows[b], sem_g[b])
            pltpu.async_copy(rwaw_hbm.at[pl.ds(base, CHUNK)], wbuf[b], sem_w[b])

        def wait_gw(b):
            pltpu.make_async_copy(hl_hbm.at[sidx[b]], rows[b], sem_g[b]).wait()
            pltpu.make_async_copy(rwaw_hbm.at[pl.ds(0, CHUNK)], wbuf[b],
                                  sem_w[b]).wait()

        def issue_scat(b):
            pltpu.async_copy(rows[b], acc.at[ridx[b]], sem_s[b], add=True)

        def wait_scat(b):
            pltpu.make_async_copy(rows[b], acc.at[ridx[b]], sem_s[b]).wait()

        def mult(b):
            def mbody(r, cc):
                for j in range(F // 16):
                    sl = pl.ds(j * 16, 16)
                    rows[b][r, sl] = rows[b][r, sl] * wbuf[b][r, sl]
                return cc
            lax.fori_loop(0, CHUNK, mbody, 0)

        issue_sidx(0, 0)
        issue_ridx(0, 0)
        wait_sidx(0)
        issue_gw(0, 0)
        issue_sidx(1, 1)

        def stage(j, b, nb):
            @pl.when(j + 1 < n)
            def _issue_next():
                wait_sidx(nb)

                @pl.when(j >= 1)
                def _():
                    wait_scat(nb)

                issue_ridx(j + 1, nb)
                issue_gw(j + 1, nb)

            wait_gw(b)

            @pl.when(j + 2 < n)
            def _prefetch_sidx():
                issue_sidx(j + 2, b)

            mult(b)
            wait_ridx(b)
            issue_scat(b)

        def pair(ph, carry):
            j0 = 2 * ph
            stage(j0, 0, 1)

            @pl.when(j0 + 1 < n)
            def _odd():
                stage(j0 + 1, 1, 0)

            return carry

        lax.fori_loop(0, (n + 1) // 2, pair, 0)
        wait_scat((n - 1) % 2)
        wait_scat((n - 2) % 2)
        plsc.subcore_barrier()

        pltpu.sync_copy(acc.at[pl.ds(r0, ra)],
                        out_hbm.at[c].at[pl.ds(r0, ra)])
        if tail:
            @pl.when(s == NS - 1)
            def _out_tail():
                pltpu.sync_copy(acc.at[pl.ds(N - tail, tail)],
                                out_hbm.at[c].at[pl.ds(N - tail, tail)])

    return k(hl, send1d, recv1d, rwaw)


def kernel(vectors, node_specie, senders, receivers, params):
    N = node_specie.shape[0]
    E = senders.shape[0]
    p = params

    vt = jnp.zeros((8, E), jnp.float32).at[0:3, :].set(vectors.T)
    spec2d = node_specie.reshape(N, 1).astype(jnp.int32)
    send1d = senders.astype(jnp.int32)
    recv1d = receivers.astype(jnp.int32)

    rwaw0, rwaw1 = _edge_weights(vt, p)
    hl0 = _node0(spec2d, p)
    aggp0 = _sc_gather_scatter(hl0, send1d, recv1d, rwaw0)
    h1, hl1, out0 = _node1(aggp0, spec2d, p)
    aggp1 = _sc_gather_scatter(hl1, send1d, recv1d, rwaw1)
    out1 = _node2(aggp1, h1, spec2d, p)
    return jnp.stack([out0, out1], axis=1)
